# Initial kernel scaffold; baseline (speedup 1.0000x reference)
import jax
import jax.numpy as jnp
from jax.experimental import pallas as pl


def kernel(gene_input, smiles_input, gene_table, smiles_table, W1, b1, W2, b2, W3, b3):
    raise NotImplementedError("write your pallas kernel here")


# speedup vs baseline: 3.4565x; 3.4565x over previous
import functools

import jax
import jax.numpy as jnp
from jax import lax
from jax.experimental import pallas as pl
from jax.experimental.pallas import tpu as pltpu
from jax.experimental.pallas import tpu_sc as plsc

B = 4096
LG = 200
LS = 200
D = 128
KG = LG * D
NW = 32
CHUNK = 128
NBUF = 2

ROWS = B * LG
RPW = ROWS // NW
NCH = RPW // CHUNK


def _phase(table, idx_hbm, out, idx_v, bufs, gsems, wsems, wid):
    base_chunk = wid * NCH
    pltpu.sync_copy(idx_hbm.at[pl.ds(base_chunk, NCH)], idx_v)

    def g_copy(slot, j):
        return pltpu.make_async_copy(
            table.at[idx_v.at[j]], bufs.at[slot], gsems[slot])

    def w_copy(slot, j):
        row0 = (base_chunk + j) * CHUNK
        return pltpu.make_async_copy(
            bufs.at[slot], out.at[pl.ds(row0, CHUNK)], wsems[slot])

    for s in range(NBUF):
        g_copy(s, s).start()

    def body(i, carry):
        for s in range(NBUF):
            j = i * NBUF + s
            g_copy(s, j).wait()
            w_copy(s, j).start()
            w_copy(s, j).wait()

            @pl.when(j + NBUF < NCH)
            def _():
                g_copy(s, j + NBUF).start()
        return carry

    lax.fori_loop(0, NCH // NBUF, body, 0)


@functools.partial(
    pl.kernel,
    out_type=(
        jax.ShapeDtypeStruct((ROWS, D), jnp.float32),
        jax.ShapeDtypeStruct((ROWS, D), jnp.float32),
    ),
    mesh=plsc.VectorSubcoreMesh(core_axis_name="c", subcore_axis_name="s"),
    scratch_types=[
        pltpu.VMEM((NCH, CHUNK), jnp.int32),
        pltpu.VMEM((NBUF, CHUNK, D), jnp.float32),
        pltpu.SemaphoreType.DMA,
        pltpu.SemaphoreType.DMA,
        pltpu.SemaphoreType.DMA,
        pltpu.SemaphoreType.DMA,
    ],
)
def _sc_gather(gene_table, gene_idx, smiles_table, smiles_idx,
               ge_out, se_out, idx_v, bufs, g0, g1, w0, w1):
    wid = lax.axis_index("c") * 16 + lax.axis_index("s")
    gsems = [g0, g1]
    wsems = [w0, w1]
    _phase(gene_table, gene_idx, ge_out, idx_v, bufs, gsems, wsems, wid)
    _phase(smiles_table, smiles_idx, se_out, idx_v, bufs, gsems, wsems, wid)


def _mlp_body(ge, se, w1g, w1s, b1, w2, b2, w3, b3, out, acc):
    k = pl.program_id(1)
    nk = pl.num_programs(1)

    @pl.when(k == 0)
    def _():
        acc[...] = jnp.zeros_like(acc)

    acc[...] += jnp.dot(ge[...], w1g[...], preferred_element_type=jnp.float32)
    acc[...] += jnp.dot(se[...], w1s[...], preferred_element_type=jnp.float32)

    @pl.when(k == nk - 1)
    def _():
        h = jnp.maximum(acc[...] + b1[...], 0.0)
        h = jnp.maximum(
            jnp.dot(h, w2[...], preferred_element_type=jnp.float32) + b2[...],
            0.0)
        x = jnp.dot(h, w3[...], preferred_element_type=jnp.float32) + b3[...]
        out[...] = 1.0 / (1.0 + jnp.exp(-x))


def _mlp(ge, se, w1g, w1s, b1, w2, b2, w3, b3):
    BB = 512
    KB = 2560
    grid = (B // BB, KG // KB)
    return pl.pallas_call(
        _mlp_body,
        grid=grid,
        in_specs=[
            pl.BlockSpec((BB, KB), lambda b, k: (b, k)),
            pl.BlockSpec((BB, KB), lambda b, k: (b, k)),
            pl.BlockSpec((KB, 64), lambda b, k: (k, 0)),
            pl.BlockSpec((KB, 64), lambda b, k: (k, 0)),
            pl.BlockSpec((1, 64), lambda b, k: (0, 0)),
            pl.BlockSpec((64, 32), lambda b, k: (0, 0)),
            pl.BlockSpec((1, 32), lambda b, k: (0, 0)),
            pl.BlockSpec((32, 1), lambda b, k: (0, 0)),
            pl.BlockSpec((1, 1), lambda b, k: (0, 0)),
        ],
        out_specs=pl.BlockSpec((BB, 1), lambda b, k: (b, 0)),
        out_shape=jax.ShapeDtypeStruct((B, 1), jnp.float32),
        scratch_shapes=[pltpu.VMEM((BB, 64), jnp.float32)],
    )(ge, se, w1g, w1s, b1, w2, b2, w3, b3)


def kernel(gene_input, smiles_input, gene_table, smiles_table,
           W1, b1, W2, b2, W3, b3):
    gidx = gene_input.reshape(ROWS // CHUNK, CHUNK)
    sidx = smiles_input.reshape(ROWS // CHUNK, CHUNK)
    ge, se = _sc_gather(gene_table, gidx, smiles_table, sidx)
    ge = ge.reshape(B, KG)
    se = se.reshape(B, KG)
    return _mlp(ge, se, W1[:KG], W1[KG:],
                b1.reshape(1, 64), W2, b2.reshape(1, 32),
                W3, b3.reshape(1, 1))


# speedup vs baseline: 4.4471x; 1.2866x over previous
import functools

import jax
import jax.numpy as jnp
from jax import lax
from jax.experimental import pallas as pl
from jax.experimental.pallas import tpu as pltpu
from jax.experimental.pallas import tpu_sc as plsc

B = 4096
LG = 200
LS = 200
D = 128
KG = LG * D
NW = 32
CHUNK = 40
CPT = LG // CHUNK
NBUF = 2
BPW = B // NW
NCH = BPW * CPT


def _phase(table, idx_v, out, bufs, gsems, wsems, b0):
    def g_copy(slot, j):
        return pltpu.make_async_copy(
            table.at[idx_v.at[j]], bufs.at[slot], gsems[slot])

    def w_copy(slot, j):
        off = pl.multiple_of((j % CPT) * CHUNK, 8)
        return pltpu.make_async_copy(
            bufs.at[slot],
            out.at[b0 + j // CPT, pl.ds(off, CHUNK)],
            wsems[slot])

    for s in range(NBUF):
        g_copy(s, s).start()

    def body(i, carry):
        for s in range(NBUF):
            j = i * NBUF + s
            g_copy(s, j).wait()
            w_copy(s, j).start()
            w_copy(s, j).wait()

            @pl.when(j + NBUF < NCH)
            def _():
                g_copy(s, j + NBUF).start()
        return carry

    lax.fori_loop(0, NCH // NBUF, body, 0)


@functools.partial(
    pl.kernel,
    out_type=(
        jax.ShapeDtypeStruct((B, LG, D), jnp.float32),
        jax.ShapeDtypeStruct((B, LS, D), jnp.float32),
    ),
    mesh=plsc.VectorSubcoreMesh(core_axis_name="c", subcore_axis_name="s"),
    scratch_types=[
        pltpu.VMEM((NCH, CHUNK), jnp.int32),
        pltpu.VMEM((NBUF, CHUNK, D), jnp.float32),
        pltpu.SemaphoreType.DMA,
        pltpu.SemaphoreType.DMA,
        pltpu.SemaphoreType.DMA,
        pltpu.SemaphoreType.DMA,
    ],
)
def _sc_gather(gene_table, gene_idx, smiles_table, smiles_idx,
               ge_out, se_out, idx_v, bufs, g0, g1, w0, w1):
    wid = lax.axis_index("c") * 16 + lax.axis_index("s")
    b0 = wid * BPW
    gsems = [g0, g1]
    wsems = [w0, w1]
    pltpu.sync_copy(gene_idx.at[pl.ds(wid * NCH, NCH)], idx_v)
    _phase(gene_table, idx_v, ge_out, bufs, gsems, wsems, b0)
    pltpu.sync_copy(smiles_idx.at[pl.ds(wid * NCH, NCH)], idx_v)
    _phase(smiles_table, idx_v, se_out, bufs, gsems, wsems, b0)


def _mlp_body(ge, se, w1g, w1s, b1, w2, b2, w3, b3, out, acc):
    k = pl.program_id(1)
    nk = pl.num_programs(1)

    @pl.when(k == 0)
    def _():
        acc[...] = jnp.zeros_like(acc)

    bb, tb, d = ge.shape
    xg = ge[...].reshape(bb, tb * d)
    xs = se[...].reshape(bb, tb * d)
    acc[...] += jnp.dot(xg, w1g[...], preferred_element_type=jnp.float32)
    acc[...] += jnp.dot(xs, w1s[...], preferred_element_type=jnp.float32)

    @pl.when(k == nk - 1)
    def _():
        h = jnp.maximum(acc[...] + b1[...], 0.0)
        h = jnp.maximum(
            jnp.dot(h, w2[...], preferred_element_type=jnp.float32) + b2[...],
            0.0)
        x = jnp.dot(h, w3[...], preferred_element_type=jnp.float32) + b3[...]
        out[...] = 1.0 / (1.0 + jnp.exp(-x))


def _mlp(ge, se, w1g, w1s, b1, w2, b2, w3, b3):
    BB = 512
    TB = 8
    KB = TB * D
    grid = (B // BB, KG // KB)
    return pl.pallas_call(
        _mlp_body,
        grid=grid,
        in_specs=[
            pl.BlockSpec((BB, TB, D), lambda b, k: (b, k, 0)),
            pl.BlockSpec((BB, TB, D), lambda b, k: (b, k, 0)),
            pl.BlockSpec((KB, 64), lambda b, k: (k, 0)),
            pl.BlockSpec((KB, 64), lambda b, k: (k, 0)),
            pl.BlockSpec((1, 64), lambda b, k: (0, 0)),
            pl.BlockSpec((64, 32), lambda b, k: (0, 0)),
            pl.BlockSpec((1, 32), lambda b, k: (0, 0)),
            pl.BlockSpec((32, 1), lambda b, k: (0, 0)),
            pl.BlockSpec((1, 1), lambda b, k: (0, 0)),
        ],
        out_specs=pl.BlockSpec((BB, 1), lambda b, k: (b, 0)),
        out_shape=jax.ShapeDtypeStruct((B, 1), jnp.float32),
        scratch_shapes=[pltpu.VMEM((BB, 64), jnp.float32)],
    )(ge, se, w1g, w1s, b1, w2, b2, w3, b3)


def kernel(gene_input, smiles_input, gene_table, smiles_table,
           W1, b1, W2, b2, W3, b3):
    gidx = gene_input.reshape(B * CPT, CHUNK)
    sidx = smiles_input.reshape(B * CPT, CHUNK)
    ge, se = _sc_gather(gene_table, gidx, smiles_table, sidx)
    return _mlp(ge, se, W1[:KG], W1[KG:],
                b1.reshape(1, 64), W2, b2.reshape(1, 32),
                W3, b3.reshape(1, 1))


# speedup vs baseline: 5.7756x; 1.2987x over previous
import functools

import jax
import jax.numpy as jnp
from jax import lax
from jax.experimental import pallas as pl
from jax.experimental.pallas import tpu as pltpu
from jax.experimental.pallas import tpu_sc as plsc

B = 4096
LG = 200
LS = 200
D = 128
KG = LG * D
NW = 32

NBG = 16
NG = B // NBG
DIM1 = NBG * LG
CHUNK = 128
CPG = DIM1 // CHUNK
GPW = NG // NW
NCH = GPW * CPG
NBUF = 2


def _phase(table, idx_v, out, bufs, gsems, wsems, g0):
    def g_copy(slot, j):
        return pltpu.make_async_copy(
            table.at[idx_v.at[j]], bufs.at[slot], gsems[slot])

    def w_copy(slot, j):
        off = pl.multiple_of((j % CPG) * CHUNK, 128)
        return pltpu.make_async_copy(
            bufs.at[slot],
            out.at[g0 + j // CPG, pl.ds(off, CHUNK)],
            wsems[slot])

    for s in range(NBUF):
        g_copy(s, s).start()

    def body(i, carry):
        for s in range(NBUF):
            j = i * NBUF + s
            g_copy(s, j).wait()
            w_copy(s, j).start()
            w_copy(s, j).wait()

            @pl.when(j + NBUF < NCH)
            def _():
                g_copy(s, j + NBUF).start()
        return carry

    lax.fori_loop(0, NCH // NBUF, body, 0)


@functools.partial(
    pl.kernel,
    out_type=(
        jax.ShapeDtypeStruct((NG, DIM1, D), jnp.float32),
        jax.ShapeDtypeStruct((NG, DIM1, D), jnp.float32),
    ),
    mesh=plsc.VectorSubcoreMesh(core_axis_name="c", subcore_axis_name="s"),
    scratch_types=[
        pltpu.VMEM((NCH, CHUNK), jnp.int32),
        pltpu.VMEM((NBUF, CHUNK, D), jnp.float32),
        pltpu.SemaphoreType.DMA,
        pltpu.SemaphoreType.DMA,
        pltpu.SemaphoreType.DMA,
        pltpu.SemaphoreType.DMA,
    ],
)
def _sc_gather(gene_table, gene_idx, smiles_table, smiles_idx,
               ge_out, se_out, idx_v, bufs, g0, g1, w0, w1):
    wid = lax.axis_index("c") * 16 + lax.axis_index("s")
    grp0 = wid * GPW
    gsems = [g0, g1]
    wsems = [w0, w1]
    pltpu.sync_copy(gene_idx.at[pl.ds(wid * NCH, NCH)], idx_v)
    _phase(gene_table, idx_v, ge_out, bufs, gsems, wsems, grp0)
    pltpu.sync_copy(smiles_idx.at[pl.ds(wid * NCH, NCH)], idx_v)
    _phase(smiles_table, idx_v, se_out, bufs, gsems, wsems, grp0)


def _mlp_body(ge, se, w1g, w1s, b1, w2, b2, w3, b3, out):
    gb = ge.shape[0]
    xg = ge[...].reshape(gb * NBG, KG)
    xs = se[...].reshape(gb * NBG, KG)
    h = jnp.dot(xg, w1g[...], preferred_element_type=jnp.float32)
    h += jnp.dot(xs, w1s[...], preferred_element_type=jnp.float32)
    h = jnp.maximum(h + b1[...], 0.0)
    h = jnp.maximum(
        jnp.dot(h, w2[...], preferred_element_type=jnp.float32) + b2[...],
        0.0)
    x = jnp.dot(h, w3[...], preferred_element_type=jnp.float32) + b3[...]
    out[...] = 1.0 / (1.0 + jnp.exp(-x))


def _mlp(ge, se, w1g, w1s, b1, w2, b2, w3, b3):
    GB = 4
    grid = (NG // GB,)
    return pl.pallas_call(
        _mlp_body,
        grid=grid,
        in_specs=[
            pl.BlockSpec((GB, DIM1, D), lambda b: (b, 0, 0)),
            pl.BlockSpec((GB, DIM1, D), lambda b: (b, 0, 0)),
            pl.BlockSpec((KG, 64), lambda b: (0, 0)),
            pl.BlockSpec((KG, 64), lambda b: (0, 0)),
            pl.BlockSpec((1, 64), lambda b: (0, 0)),
            pl.BlockSpec((64, 32), lambda b: (0, 0)),
            pl.BlockSpec((1, 32), lambda b: (0, 0)),
            pl.BlockSpec((32, 1), lambda b: (0, 0)),
            pl.BlockSpec((1, 1), lambda b: (0, 0)),
        ],
        out_specs=pl.BlockSpec((GB * NBG, 1), lambda b: (b, 0)),
        out_shape=jax.ShapeDtypeStruct((B, 1), jnp.float32),
    )(ge, se, w1g, w1s, b1, w2, b2, w3, b3)


def kernel(gene_input, smiles_input, gene_table, smiles_table,
           W1, b1, W2, b2, W3, b3):
    gidx = gene_input.reshape(B * LG // CHUNK, CHUNK)
    sidx = smiles_input.reshape(B * LS // CHUNK, CHUNK)
    ge, se = _sc_gather(gene_table, gidx, smiles_table, sidx)
    return _mlp(ge, se, W1[:KG], W1[KG:],
                b1.reshape(1, 64), W2, b2.reshape(1, 32),
                W3, b3.reshape(1, 1))


# speedup vs baseline: 7.4393x; 1.2881x over previous
import functools

import jax
import jax.numpy as jnp
from jax import lax
from jax.experimental import pallas as pl
from jax.experimental.pallas import tpu as pltpu
from jax.experimental.pallas import tpu_sc as plsc

B = 4096
LG = 200
LS = 200
D = 128
KG = LG * D
NW = 32

NBG = 16
NG = B // NBG
DIM1 = NBG * LG
CHUNK = 128
CPG = DIM1 // CHUNK
GPW = NG // NW
NCH = GPW * CPG
NBUF = 2

SV = 1024
CH_S = 100
BPW = B // NW
NCH_S = BPW * 2


def _gene_phase(table, idx_v, out, bufs, gsems, wsems, g0):
    def g_copy(slot, j):
        return pltpu.make_async_copy(
            table.at[idx_v.at[j]], bufs.at[slot], gsems[slot])

    def w_copy(slot, j):
        off = pl.multiple_of((j % CPG) * CHUNK, 128)
        return pltpu.make_async_copy(
            bufs.at[slot],
            out.at[g0 + j // CPG, pl.ds(off, CHUNK)],
            wsems[slot])

    for s in range(NBUF):
        g_copy(s, s).start()

    def body(i, carry):
        for s in range(NBUF):
            j = i * NBUF + s
            g_copy(s, j).wait()
            w_copy(s, j).start()
            w_copy(s, j).wait()

            @pl.when(j + NBUF < NCH)
            def _():
                g_copy(s, j + NBUF).start()
        return carry

    lax.fori_loop(0, NCH // NBUF, body, 0)


def _smiles_phase(p_tab, idx_v, sp_out, bufs, gsems, accbuf, b0):
    def g_copy(slot, j):
        return pltpu.make_async_copy(
            p_tab.at[idx_v.at[j]], bufs.at[slot, pl.ds(0, CH_S)],
            gsems[slot])

    def _sum_chunk(slot, acc):
        def rbody(r, a):
            return (a[0] + bufs[slot, r, pl.ds(0, 16)],
                    a[1] + bufs[slot, r, pl.ds(16, 16)],
                    a[2] + bufs[slot, r, pl.ds(32, 16)],
                    a[3] + bufs[slot, r, pl.ds(48, 16)])
        return lax.fori_loop(0, CH_S, rbody, acc)

    for s in range(NBUF):
        g_copy(s, s).start()

    def body(i, carry):
        zero = jnp.zeros((16,), jnp.float32)
        acc = (zero, zero, zero, zero)
        for s in range(NBUF):
            j = 2 * i + s
            g_copy(s, j).wait()

            @pl.when(j + NBUF < NCH_S)
            def _():
                g_copy(s, j + NBUF).start()

            acc = _sum_chunk(s, acc)
        for q in range(4):
            accbuf[i, pl.ds(16 * q, 16)] = acc[q]
        return carry

    lax.fori_loop(0, NCH_S // 2, body, 0)
    pltpu.sync_copy(accbuf, sp_out.at[pl.ds(b0, BPW)])


@functools.partial(
    pl.kernel,
    out_type=(
        jax.ShapeDtypeStruct((NG, DIM1, D), jnp.float32),
        jax.ShapeDtypeStruct((B, 64), jnp.float32),
    ),
    mesh=plsc.VectorSubcoreMesh(core_axis_name="c", subcore_axis_name="s"),
    scratch_types=[
        pltpu.VMEM((NCH, CHUNK), jnp.int32),
        pltpu.VMEM((NCH_S, CH_S), jnp.int32),
        pltpu.VMEM((NBUF, CHUNK, D), jnp.float32),
        pltpu.VMEM((BPW, 64), jnp.float32),
        pltpu.SemaphoreType.DMA,
        pltpu.SemaphoreType.DMA,
        pltpu.SemaphoreType.DMA,
        pltpu.SemaphoreType.DMA,
    ],
)
def _sc_gather(gene_table, gene_idx, p_tab, smiles_idx,
               ge_out, sp_out, idx_v, idx_s, bufs, accbuf,
               g0, g1, w0, w1):
    wid = lax.axis_index("c") * 16 + lax.axis_index("s")
    gsems = [g0, g1]
    wsems = [w0, w1]
    pltpu.sync_copy(gene_idx.at[pl.ds(wid * NCH, NCH)], idx_v)
    _gene_phase(gene_table, idx_v, ge_out, bufs, gsems, wsems, wid * GPW)
    pltpu.sync_copy(smiles_idx.at[pl.ds(wid * NCH_S, NCH_S)], idx_s)
    _smiles_phase(p_tab, idx_s, sp_out, bufs, gsems, accbuf, wid * BPW)


def _p_body(st, w1s, out):
    out[...] = jnp.dot(st[...], w1s[...], preferred_element_type=jnp.float32)


def _p_compute(st_pad, w1s):
    return pl.pallas_call(
        _p_body,
        grid=(LS,),
        in_specs=[
            pl.BlockSpec((SV, D), lambda t: (0, 0)),
            pl.BlockSpec((D, D), lambda t: (t, 0)),
        ],
        out_specs=pl.BlockSpec((SV, D), lambda t: (t, 0)),
        out_shape=jax.ShapeDtypeStruct((LS * SV, D), jnp.float32),
    )(st_pad, w1s)


def _mlp_body(ge, sp, w1g, b1, w2, b2, w3, b3, out):
    gb = ge.shape[0]
    xg = ge[...].reshape(gb * NBG, KG)
    h = jnp.dot(xg, w1g[...], preferred_element_type=jnp.float32)
    h = jnp.maximum(h + sp[...] + b1[...], 0.0)
    h = jnp.maximum(
        jnp.dot(h, w2[...], preferred_element_type=jnp.float32) + b2[...],
        0.0)
    x = jnp.dot(h, w3[...], preferred_element_type=jnp.float32) + b3[...]
    out[...] = 1.0 / (1.0 + jnp.exp(-x))


def _mlp(ge, sp, w1g, b1, w2, b2, w3, b3):
    GB = 8
    grid = (NG // GB,)
    return pl.pallas_call(
        _mlp_body,
        grid=grid,
        in_specs=[
            pl.BlockSpec((GB, DIM1, D), lambda b: (b, 0, 0)),
            pl.BlockSpec((GB * NBG, 64), lambda b: (b, 0)),
            pl.BlockSpec((KG, 64), lambda b: (0, 0)),
            pl.BlockSpec((1, 64), lambda b: (0, 0)),
            pl.BlockSpec((64, 32), lambda b: (0, 0)),
            pl.BlockSpec((1, 32), lambda b: (0, 0)),
            pl.BlockSpec((32, 1), lambda b: (0, 0)),
            pl.BlockSpec((1, 1), lambda b: (0, 0)),
        ],
        out_specs=pl.BlockSpec((GB * NBG, 1), lambda b: (b, 0)),
        out_shape=jax.ShapeDtypeStruct((B, 1), jnp.float32),
    )(ge, sp, w1g, b1, w2, b2, w3, b3)


def kernel(gene_input, smiles_input, gene_table, smiles_table,
           W1, b1, W2, b2, W3, b3):
    gidx = gene_input.reshape(B * LG // CHUNK, CHUNK)
    st_pad = jnp.pad(smiles_table, ((0, SV - smiles_table.shape[0]), (0, 0)))
    w1s_pad = jnp.pad(W1[KG:], ((0, 0), (0, D - 64)))
    p_tab = _p_compute(st_pad, w1s_pad)
    sidx = (smiles_input
            + jnp.arange(LS, dtype=jnp.int32)[None, :] * SV)
    sidx = sidx.reshape(B * 2, CH_S)
    ge, sp = _sc_gather(gene_table, gidx, p_tab, sidx)
    return _mlp(ge, sp, W1[:KG],
                b1.reshape(1, 64), W2, b2.reshape(1, 32),
                W3, b3.reshape(1, 1))
